# baseline (speedup 1.0000x reference)
import functools

import jax
import jax.numpy as jnp
from jax import lax
from jax.experimental import pallas as pl
from jax.experimental.pallas import tpu as pltpu
from jax.experimental.pallas import tpu_sc as plsc

NW = 32
CHUNK = 1024
BV = 32768
TB = 16384


def _eye(n):
    r = lax.broadcasted_iota(jnp.int32, (n, n), 0)
    c = lax.broadcasted_iota(jnp.int32, (n, n), 1)
    return (r == c).astype(jnp.float32)


SUB = 2048


def _tc_normalize_body(emb_t_ref, out_ref):
    d_dim = emb_t_ref.shape[0]
    eye = _eye(d_dim)
    for j in range(BV // SUB):
        x = emb_t_ref[:, j * SUB:(j + 1) * SUB]
        s = jnp.sum(x * x, axis=0)
        z = x * lax.rsqrt(s)[None, :]
        out_ref[j * SUB:(j + 1) * SUB, :] = lax.dot_general(
            z, eye, (((0,), (0,)), ((), ())),
            preferred_element_type=jnp.float32)


def _normalized_table(emb_t, vocab, d_dim):
    grid = (vocab + BV - 1) // BV
    return pl.pallas_call(
        _tc_normalize_body,
        grid=(grid,),
        in_specs=[pl.BlockSpec((d_dim, BV), lambda i: (0, i))],
        out_specs=pl.BlockSpec((BV, d_dim), lambda i: (i, 0)),
        out_shape=jax.ShapeDtypeStruct((grid * BV, d_dim), jnp.float32),
    )(emb_t)


def _tc_layout_body(g_ref, out_ref):
    x = g_ref[...]
    d_dim = x.shape[1]
    t = lax.dot_general(
        _eye(d_dim), x, (((1,), (1,)), ((), ())),
        preferred_element_type=jnp.float32)
    out_ref[...] = t[None]


def _to_output_layout(flat, f_dim, b_dim, d_dim):
    nbk = b_dim // TB
    return pl.pallas_call(
        _tc_layout_body,
        grid=(f_dim, nbk),
        in_specs=[pl.BlockSpec((TB, d_dim), lambda f, k: (f * nbk + k, 0))],
        out_specs=pl.BlockSpec((1, d_dim, TB), lambda f, k: (f, 0, k)),
        out_shape=jax.ShapeDtypeStruct((f_dim, d_dim, b_dim), jnp.float32),
    )(flat)


def kernel(indices, embeddings):
    b_dim, f_dim = indices.shape
    vocab, d_dim = embeddings.shape
    flat_b = indices.size
    assert flat_b % (NW * CHUNK) == 0
    b_per_w = flat_b // NW
    n_chunks = b_per_w // CHUNK

    table_n = _normalized_table(embeddings.T, vocab, d_dim)

    idx2 = indices.T.astype(jnp.int32).reshape(NW, b_per_w)

    mesh = plsc.VectorSubcoreMesh(core_axis_name="c", subcore_axis_name="s")

    @functools.partial(
        pl.kernel,
        mesh=mesh,
        compiler_params=pltpu.CompilerParams(
            needs_layout_passes=False, use_tc_tiling_on_sc=False),
        out_type=jax.ShapeDtypeStruct((flat_b, d_dim), jnp.float32),
        scratch_types=[
            pltpu.VMEM((b_per_w,), jnp.int32),
            pltpu.VMEM((CHUNK, d_dim), jnp.float32),
            pltpu.VMEM((CHUNK, d_dim), jnp.float32),
            pltpu.SemaphoreType.DMA,
            pltpu.SemaphoreType.DMA,
        ],
    )
    def run(table_hbm, idx_hbm, out_hbm, idx_v, buf0, buf1, sem0, sem1):
        wid = lax.axis_index("s") * 2 + lax.axis_index("c")
        pltpu.sync_copy(idx_hbm.at[wid], idx_v)
        bufs, sems = (buf0, buf1), (sem0, sem1)

        pend = [None, None]
        pend[0] = pltpu.async_copy(
            table_hbm.at[idx_v.at[pl.ds(0, CHUNK)]], bufs[0], sems[0])
        for c in range(n_chunks):
            cur = c % 2
            if c + 1 < n_chunks:
                nxt = (c + 1) % 2
                pend[nxt] = pltpu.async_copy(
                    table_hbm.at[idx_v.at[pl.ds((c + 1) * CHUNK, CHUNK)]],
                    bufs[nxt], sems[nxt])
            pend[cur].wait()
            base = wid * b_per_w + c * CHUNK
            pltpu.sync_copy(bufs[cur], out_hbm.at[pl.ds(base, CHUNK)])

    out = run(table_n, idx2)
    return out.reshape(f_dim, b_dim, d_dim).transpose(1, 0, 2)


# speedup vs baseline: 1.5789x; 1.5789x over previous
import functools

import jax
import jax.numpy as jnp
from jax import lax
from jax.experimental import pallas as pl
from jax.experimental.pallas import tpu as pltpu
from jax.experimental.pallas import tpu_sc as plsc

NW = 32
CHUNK = 1024
BV = 32768
TB = 16384


def _eye(n):
    r = lax.broadcasted_iota(jnp.int32, (n, n), 0)
    c = lax.broadcasted_iota(jnp.int32, (n, n), 1)
    return (r == c).astype(jnp.float32)


SUB = 2048


def _tc_normalize_body(emb_t_ref, out_ref):
    d_dim = emb_t_ref.shape[0]
    lanes = 128 // d_dim
    eye = _eye(d_dim)
    for j in range(BV // SUB):
        x = emb_t_ref[:, j * SUB:(j + 1) * SUB]
        s = jnp.sum(x * x, axis=0)
        z = x * lax.rsqrt(s)[None, :]
        t = lax.dot_general(
            z, eye, (((0,), (0,)), ((), ())),
            preferred_element_type=jnp.float32)
        q, k = j // lanes, j % lanes
        out_ref[q * SUB:(q + 1) * SUB, k * d_dim:(k + 1) * d_dim] = t


def _normalized_table(emb_t, vocab, d_dim):
    grid = (vocab + BV - 1) // BV
    rows = BV * d_dim // 128
    return pl.pallas_call(
        _tc_normalize_body,
        grid=(grid,),
        in_specs=[pl.BlockSpec((d_dim, BV), lambda i: (0, i))],
        out_specs=pl.BlockSpec((rows, 128), lambda i: (i, 0)),
        out_shape=jax.ShapeDtypeStruct((grid * rows, 128), jnp.float32),
    )(emb_t)


def _tc_layout_body(g_ref, out_ref):
    x = g_ref[...]
    d_dim = x.shape[1]
    t = lax.dot_general(
        _eye(d_dim), x, (((1,), (1,)), ((), ())),
        preferred_element_type=jnp.float32)
    out_ref[...] = t[None]


def _to_output_layout(flat, f_dim, b_dim, d_dim):
    nbk = b_dim // TB
    return pl.pallas_call(
        _tc_layout_body,
        grid=(f_dim, nbk),
        in_specs=[pl.BlockSpec((TB, d_dim), lambda f, k: (f * nbk + k, 0))],
        out_specs=pl.BlockSpec((1, d_dim, TB), lambda f, k: (f, 0, k)),
        out_shape=jax.ShapeDtypeStruct((f_dim, d_dim, b_dim), jnp.float32),
    )(flat)


def kernel(indices, embeddings):
    b_dim, f_dim = indices.shape
    vocab, d_dim = embeddings.shape
    flat_b = indices.size
    assert flat_b % (NW * CHUNK) == 0
    b_per_w = flat_b // NW
    n_chunks = b_per_w // CHUNK

    flat_table = _normalized_table(embeddings.T, vocab, d_dim)
    table_n = flat_table.reshape(flat_table.shape[0] * (128 // d_dim), d_dim)

    lanes = 128 // d_dim
    v = indices.T.astype(jnp.int32)
    i_, rem = v // BV, v % BV
    j, u = rem // SUB, rem % SUB
    vrow = (i_ * (BV // lanes) + (j // lanes) * SUB + u) * lanes + (j % lanes)
    idx2 = vrow.reshape(NW, b_per_w)

    mesh = plsc.VectorSubcoreMesh(core_axis_name="c", subcore_axis_name="s")

    @functools.partial(
        pl.kernel,
        mesh=mesh,
        compiler_params=pltpu.CompilerParams(
            needs_layout_passes=False, use_tc_tiling_on_sc=False),
        out_type=jax.ShapeDtypeStruct((flat_b, d_dim), jnp.float32),
        scratch_types=[
            pltpu.VMEM((b_per_w,), jnp.int32),
            pltpu.VMEM((CHUNK, d_dim), jnp.float32),
            pltpu.VMEM((CHUNK, d_dim), jnp.float32),
            pltpu.SemaphoreType.DMA,
            pltpu.SemaphoreType.DMA,
        ],
    )
    def run(table_hbm, idx_hbm, out_hbm, idx_v, buf0, buf1, sem0, sem1):
        wid = lax.axis_index("s") * 2 + lax.axis_index("c")
        pltpu.sync_copy(idx_hbm.at[wid], idx_v)
        bufs, sems = (buf0, buf1), (sem0, sem1)

        pend = [None, None]
        pend[0] = pltpu.async_copy(
            table_hbm.at[idx_v.at[pl.ds(0, CHUNK)]], bufs[0], sems[0])
        for c in range(n_chunks):
            cur = c % 2
            if c + 1 < n_chunks:
                nxt = (c + 1) % 2
                pend[nxt] = pltpu.async_copy(
                    table_hbm.at[idx_v.at[pl.ds((c + 1) * CHUNK, CHUNK)]],
                    bufs[nxt], sems[nxt])
            pend[cur].wait()
            base = wid * b_per_w + c * CHUNK
            pltpu.sync_copy(bufs[cur], out_hbm.at[pl.ds(base, CHUNK)])

    out = run(table_n, idx2)
    return out.reshape(f_dim, b_dim, d_dim).transpose(1, 0, 2)
